# Initial kernel scaffold; baseline (speedup 1.0000x reference)
import jax
import jax.numpy as jnp
from jax.experimental import pallas as pl


def kernel(emissions, tags, mask, transitions, inc_idx, inc_mask):
    raise NotImplementedError("write your pallas kernel here")


# speedup vs baseline: 5.2995x; 5.2995x over previous
import functools

import jax
import jax.numpy as jnp
from jax.experimental import pallas as pl
from jax.experimental.pallas import tpu as pltpu

FORBID = -10000.0


def _crf_kernel(emis_ref, tags_ref, maskf_ref, idxT_ref, mskT_ref, trT_ref,
                out_ref, alpha_scr, gold_scr, ptag_scr,
                *, TB, NT, B, K, M):
    it = pl.program_id(1)
    ib = pl.program_id(0)

    idxT = idxT_ref[...]
    mskT = mskT_ref[...]
    trT = trT_ref[...]

    iota_i = jax.lax.broadcasted_iota(jnp.int32, (K, K), 0)
    W2 = jnp.zeros((K, K), jnp.float32)
    Tlog2 = jnp.full((K, K), FORBID, jnp.float32)
    for m in range(M):
        eqm = (iota_i == idxT[m:m + 1, :]) & (mskT[m:m + 1, :] > 0)
        W2 = W2 + jnp.where(eqm, jnp.exp(trT[m:m + 1, :]), 0.0)
        Tlog2 = jnp.where(eqm, jnp.maximum(Tlog2, trT[m:m + 1, :]), Tlog2)

    BB = emis_ref.shape[0]
    iota_k = jax.lax.broadcasted_iota(jnp.int32, (BB, K), 1)

    def gather_k(vals, tag):
        return jnp.sum(jnp.where(iota_k == tag, vals, 0.0), axis=1,
                       keepdims=True)

    def step(alpha, gold, ptag, tt):
        e_t = emis_ref[:, tt, :]
        tag_t = tags_ref[:, tt, :]
        mk = maskf_ref[:, tt, :]
        rowmax = jnp.max(alpha, axis=1, keepdims=True)
        p = jnp.exp(alpha - rowmax)
        s = jnp.dot(p, W2, preferred_element_type=jnp.float32)
        na = jnp.log(s) + rowmax + e_t
        alpha = jnp.where(mk > 0.0, na, alpha)
        emit_g = gather_k(e_t, tag_t)
        oh_prev = jnp.where(iota_k == ptag, 1.0, 0.0)
        rows = jnp.dot(oh_prev, Tlog2, preferred_element_type=jnp.float32)
        trans_g = gather_k(rows, tag_t)
        gold = gold + mk * (emit_g + trans_g)
        return alpha, gold, tag_t

    alpha_c = alpha_scr[...]
    gold_c = gold_scr[...]
    ptag_c = ptag_scr[...]
    a_stp, g_stp, t_stp = step(alpha_c, gold_c, ptag_c, 0)
    e0 = emis_ref[:, 0, :]
    tag0 = tags_ref[:, 0, :]
    first = (it == 0)
    alpha = jnp.where(first, e0, a_stp)
    gold = jnp.where(first, gather_k(e0, tag0), g_stp)
    ptag = jnp.where(first, tag0, t_stp)

    for tt in range(1, TB):
        alpha, gold, ptag = step(alpha, gold, ptag, tt)

    alpha_scr[...] = alpha
    gold_scr[...] = gold
    ptag_scr[...] = ptag

    @pl.when(jnp.logical_and(ib == 0, it == 0))
    def _init():
        out_ref[...] = jnp.zeros((1, 1), jnp.float32)

    @pl.when(it == NT - 1)
    def _fin():
        rowmax = jnp.max(alpha, axis=1, keepdims=True)
        logZ = jnp.log(jnp.sum(jnp.exp(alpha - rowmax), axis=1,
                               keepdims=True)) + rowmax
        out_ref[...] += jnp.sum(logZ - gold).reshape(1, 1) / B


def kernel(emissions, tags, mask, transitions, inc_idx, inc_mask):
    B, T, K = emissions.shape
    M = inc_idx.shape[1]
    BB = 256
    TB = 40
    NB = B // BB
    NT = T // TB

    tags3 = tags.reshape(B, T, 1)
    maskf3 = mask.astype(jnp.float32).reshape(B, T, 1)
    idxT = inc_idx.T
    mskT = inc_mask.T.astype(jnp.int32)
    trT = transitions.T

    grid = (NB, NT)
    kfn = functools.partial(_crf_kernel, TB=TB, NT=NT, B=B, K=K, M=M)
    out = pl.pallas_call(
        kfn,
        grid=grid,
        in_specs=[
            pl.BlockSpec((BB, TB, K), lambda ib, it: (ib, it, 0)),
            pl.BlockSpec((BB, TB, 1), lambda ib, it: (ib, it, 0)),
            pl.BlockSpec((BB, TB, 1), lambda ib, it: (ib, it, 0)),
            pl.BlockSpec((M, K), lambda ib, it: (0, 0)),
            pl.BlockSpec((M, K), lambda ib, it: (0, 0)),
            pl.BlockSpec((M, K), lambda ib, it: (0, 0)),
        ],
        out_specs=pl.BlockSpec((1, 1), lambda ib, it: (0, 0)),
        out_shape=jax.ShapeDtypeStruct((1, 1), jnp.float32),
        scratch_shapes=[
            pltpu.VMEM((BB, K), jnp.float32),
            pltpu.VMEM((BB, 1), jnp.float32),
            pltpu.VMEM((BB, 1), jnp.int32),
        ],
    )(emissions, tags3, maskf3, idxT, mskT, trT)
    return out[0, 0]
